# initial kernel scaffold (unmeasured)
import jax
import jax.numpy as jnp
from jax import lax
from jax.experimental import pallas as pl
from jax.experimental.pallas import tpu as pltpu

N_DEV = 4
MC = 2048
K = 2048
N = 4096
NH = N // 2
NT = 1024

_PRECISION = lax.Precision.HIGHEST


def _gelu(y):
    c = 0.7978845608028654
    return 0.5 * y * (1.0 + jnp.tanh(c * (y + 0.044715 * y * y * y)))


def kernel(x, w_mat):
    def body(x_ref, w_ref, out_ref,
             acc_r, acc_l, rcv_r, rcv_l,
             xv, wv, cv, rv,
             copy_sem, send_sems, recv_sems):
        me = lax.axis_index("i")
        right = lax.rem(me + 1, N_DEV)
        left = lax.rem(me + N_DEV - 1, N_DEV)

        barrier = pltpu.get_barrier_semaphore()
        for nbr in (left, right):
            pl.semaphore_signal(barrier, inc=1, device_id=(nbr,),
                                device_id_type=pl.DeviceIdType.MESH)
        pl.semaphore_wait(barrier, 2)

        def local_copy(src, dst):
            c = pltpu.make_async_copy(src, dst, copy_sem)
            c.start()
            c.wait()

        sends = []
        for t in range(N_DEV):
            for f in range(2):
                if f == 0:
                    chunk = lax.rem(me + (N_DEV - 1 - t), N_DEV)
                    acc, rcv, col0, dst = acc_r, rcv_r, 0, right
                else:
                    chunk = lax.rem(me + 1 + t, N_DEV)
                    acc, rcv, col0, dst = acc_l, rcv_l, NH, left

                local_copy(x_ref.at[pl.ds(chunk * MC, MC), :], xv)

                if t > 0:
                    pltpu.make_async_remote_copy(
                        src_ref=acc.at[t - 1], dst_ref=rcv.at[t - 1],
                        send_sem=send_sems.at[f, t - 1],
                        recv_sem=recv_sems.at[f, t - 1],
                        device_id=(dst,),
                        device_id_type=pl.DeviceIdType.MESH,
                    ).wait_recv()

                for j in range(NH // NT):
                    local_copy(w_ref.at[:, pl.ds(col0 + j * NT, NT)], wv)
                    part = lax.dot_general(
                        xv[...], wv[...], (((1,), (0,)), ((), ())),
                        preferred_element_type=jnp.float32,
                        precision=_PRECISION)
                    if t > 0:
                        local_copy(rcv.at[t - 1, :, pl.ds(j * NT, NT)], rv)
                        part = part + rv[...]
                    if t == N_DEV - 1:
                        cv[...] = _gelu(part)
                        local_copy(cv, out_ref.at[:, pl.ds(col0 + j * NT, NT)])
                    else:
                        cv[...] = part
                        local_copy(cv, acc.at[t, :, pl.ds(j * NT, NT)])

                if t < N_DEV - 1:
                    rdma = pltpu.make_async_remote_copy(
                        src_ref=acc.at[t], dst_ref=rcv.at[t],
                        send_sem=send_sems.at[f, t],
                        recv_sem=recv_sems.at[f, t],
                        device_id=(dst,),
                        device_id_type=pl.DeviceIdType.MESH)
                    rdma.start()
                    sends.append(rdma)

        for rdma in sends:
            rdma.wait_send()

    return pl.pallas_call(
        body,
        out_shape=jax.ShapeDtypeStruct((MC, N), jnp.float32),
        in_specs=[
            pl.BlockSpec(memory_space=pl.ANY),
            pl.BlockSpec(memory_space=pl.ANY),
        ],
        out_specs=pl.BlockSpec(memory_space=pl.ANY),
        scratch_shapes=[
            pltpu.HBM((N_DEV - 1, MC, NH), jnp.float32),
            pltpu.HBM((N_DEV - 1, MC, NH), jnp.float32),
            pltpu.HBM((N_DEV - 1, MC, NH), jnp.float32),
            pltpu.HBM((N_DEV - 1, MC, NH), jnp.float32),
            pltpu.VMEM((MC, K), jnp.float32),
            pltpu.VMEM((K, NT), jnp.float32),
            pltpu.VMEM((MC, NT), jnp.float32),
            pltpu.VMEM((MC, NT), jnp.float32),
            pltpu.SemaphoreType.DMA,
            pltpu.SemaphoreType.DMA((2, N_DEV - 1)),
            pltpu.SemaphoreType.DMA((2, N_DEV - 1)),
        ],
        compiler_params=pltpu.CompilerParams(collective_id=0),
    )(x, w_mat)


# baseline (device time: 808674 ns/iter reference)
import jax
import jax.numpy as jnp
from jax import lax
from jax.experimental import pallas as pl
from jax.experimental.pallas import tpu as pltpu

N_DEV = 4
MC = 2048
K = 2048
N = 4096
NH = N // 2
NT = 1024
MT = 1024


def _gelu(y):
    c = 0.7978845608028654
    return 0.5 * y * (1.0 + jnp.tanh(c * (y + 0.044715 * y * y * y)))


def kernel(x, w_mat):
    def body(x_ref, w_ref, out_ref,
             acc_r, acc_l, rcv_r, rcv_l,
             xv, xb, wv, cv, rv,
             copy_sem, send_sems, recv_sems):
        me = lax.axis_index("i")
        right = lax.rem(me + 1, N_DEV)
        left = lax.rem(me + N_DEV - 1, N_DEV)

        barrier = pltpu.get_barrier_semaphore()
        for nbr in (left, right):
            pl.semaphore_signal(barrier, inc=1, device_id=(nbr,),
                                device_id_type=pl.DeviceIdType.MESH)
        pl.semaphore_wait(barrier, 2)

        def local_copy(src, dst):
            c = pltpu.make_async_copy(src, dst, copy_sem)
            c.start()
            c.wait()

        sends = []
        for t in range(N_DEV):
            for f in range(2):
                if f == 0:
                    chunk = lax.rem(me + (N_DEV - 1 - t), N_DEV)
                    acc, rcv, col0, dst = acc_r, rcv_r, 0, right
                else:
                    chunk = lax.rem(me + 1 + t, N_DEV)
                    acc, rcv, col0, dst = acc_l, rcv_l, NH, left

                for mi in range(MC // MT):
                    local_copy(
                        x_ref.at[pl.ds(chunk * MC + mi * MT, MT), :], xv)
                    xb[mi * MT:(mi + 1) * MT, :] = xv[...].astype(jnp.bfloat16)

                if t > 0:
                    pltpu.make_async_remote_copy(
                        src_ref=acc.at[t - 1], dst_ref=rcv.at[t - 1],
                        send_sem=send_sems.at[f, t - 1],
                        recv_sem=recv_sems.at[f, t - 1],
                        device_id=(dst,),
                        device_id_type=pl.DeviceIdType.MESH,
                    ).wait_recv()

                for j in range(NH // NT):
                    local_copy(w_ref.at[:, pl.ds(col0 + j * NT, NT)], wv)
                    wb = wv[...].astype(jnp.bfloat16)
                    for mi in range(MC // MT):
                        part = lax.dot_general(
                            xb[mi * MT:(mi + 1) * MT, :], wb,
                            (((1,), (0,)), ((), ())),
                            preferred_element_type=jnp.float32)
                        if t > 0:
                            local_copy(
                                rcv.at[t - 1, pl.ds(mi * MT, MT),
                                       pl.ds(j * NT, NT)], rv)
                            part = part + rv[...]
                        if t == N_DEV - 1:
                            cv[...] = _gelu(part)
                            local_copy(cv, out_ref.at[
                                pl.ds(mi * MT, MT),
                                pl.ds(col0 + j * NT, NT)])
                        else:
                            cv[...] = part
                            local_copy(cv, acc.at[
                                t, pl.ds(mi * MT, MT), pl.ds(j * NT, NT)])

                if t < N_DEV - 1:
                    rdma = pltpu.make_async_remote_copy(
                        src_ref=acc.at[t], dst_ref=rcv.at[t],
                        send_sem=send_sems.at[f, t],
                        recv_sem=recv_sems.at[f, t],
                        device_id=(dst,),
                        device_id_type=pl.DeviceIdType.MESH)
                    rdma.start()
                    sends.append(rdma)

        for rdma in sends:
            rdma.wait_send()

    comm_buf = jax.ShapeDtypeStruct((N_DEV - 1, MC, NH), jnp.float32)
    out, *_ = pl.pallas_call(
        body,
        out_shape=[
            jax.ShapeDtypeStruct((MC, N), jnp.float32),
            comm_buf,
            comm_buf,
            comm_buf,
            comm_buf,
        ],
        in_specs=[
            pl.BlockSpec(memory_space=pl.ANY),
            pl.BlockSpec(memory_space=pl.ANY),
        ],
        out_specs=[pl.BlockSpec(memory_space=pl.ANY)] * 5,
        scratch_shapes=[
            pltpu.VMEM((MT, K), jnp.float32),
            pltpu.VMEM((MC, K), jnp.bfloat16),
            pltpu.VMEM((K, NT), jnp.float32),
            pltpu.VMEM((MT, NT), jnp.float32),
            pltpu.VMEM((MT, NT), jnp.float32),
            pltpu.SemaphoreType.DMA,
            pltpu.SemaphoreType.DMA((2, N_DEV - 1)),
            pltpu.SemaphoreType.DMA((2, N_DEV - 1)),
        ],
        compiler_params=pltpu.CompilerParams(
            collective_id=0,
            vmem_limit_bytes=60 * 1024 * 1024,
        ),
    )(x, w_mat)
    return out


# device time: 619823 ns/iter; 1.3047x vs baseline; 1.3047x over previous
import jax
import jax.numpy as jnp
from jax import lax
from jax.experimental import pallas as pl
from jax.experimental.pallas import tpu as pltpu

N_DEV = 4
MC = 2048
K = 2048
N = 4096
NH = N // 2
NT = 1024
MT = 1024

COMM_DTYPE = jnp.bfloat16


def _gelu(y):
    c = 0.7978845608028654
    return 0.5 * y * (1.0 + jnp.tanh(c * (y + 0.044715 * y * y * y)))


def kernel(x, w_mat):
    def body(x_ref, w_ref, out_ref,
             acc_r, acc_l, rcv_r, rcv_l,
             xv, xb, wv, cv, cc, av, rv,
             copy_sem, ld_sems, send_sems, recv_sems):
        me = lax.axis_index("i")
        right = lax.rem(me + 1, N_DEV)
        left = lax.rem(me + N_DEV - 1, N_DEV)

        barrier = pltpu.get_barrier_semaphore()
        for nbr in (left, right):
            pl.semaphore_signal(barrier, inc=1, device_id=(nbr,),
                                device_id_type=pl.DeviceIdType.MESH)
        pl.semaphore_wait(barrier, 2)

        def local_copy(src, dst):
            c = pltpu.make_async_copy(src, dst, copy_sem)
            c.start()
            c.wait()

        def flow(f, t):
            if f == 0:
                return acc_r, rcv_r, 0, right, lax.rem(me + N_DEV - 1 - t, N_DEV)
            return acc_l, rcv_l, NH, left, lax.rem(me + 1 + t, N_DEV)

        def load_chunk(chunk):
            for mi in range(MC // MT):
                local_copy(x_ref.at[pl.ds(chunk * MC + mi * MT, MT), :], xv)
                xb[mi * MT:(mi + 1) * MT, :] = xv[...].astype(jnp.bfloat16)

        def compute_tiles(f, t):
            acc, _, col0, _, _ = flow(f, t)
            for j in range(NH // NT):
                local_copy(w_ref.at[:, pl.ds(col0 + j * NT, NT)], wv)
                wb = wv[...].astype(jnp.bfloat16)
                for mi in range(MC // MT):
                    part = lax.dot_general(
                        xb[mi * MT:(mi + 1) * MT, :], wb,
                        (((1,), (0,)), ((), ())),
                        preferred_element_type=jnp.float32)
                    cc[...] = part.astype(COMM_DTYPE)
                    local_copy(cc, acc.at[2 * t + j, pl.ds(mi * MT, MT), :])

        sends = []

        def send_hop(f, t):
            acc, rcv, _, dst, _ = flow(f, t)
            rdma = pltpu.make_async_remote_copy(
                src_ref=acc.at[pl.ds(2 * t, 2)],
                dst_ref=rcv.at[pl.ds(2 * t, 2)],
                send_sem=send_sems.at[f, t],
                recv_sem=recv_sems.at[f, t],
                device_id=(dst,),
                device_id_type=pl.DeviceIdType.MESH)
            rdma.start()
            sends.append(rdma)

        def wait_hop(f, t):
            acc, rcv, _, dst, _ = flow(f, t)
            pltpu.make_async_remote_copy(
                src_ref=acc.at[pl.ds(2 * t, 2)],
                dst_ref=rcv.at[pl.ds(2 * t, 2)],
                send_sem=send_sems.at[f, t],
                recv_sem=recv_sems.at[f, t],
                device_id=(dst,),
                device_id_type=pl.DeviceIdType.MESH).wait_recv()

        def load2(src_a, src_b, dst_a, dst_b):
            ca = pltpu.make_async_copy(src_a, dst_a, ld_sems.at[0])
            cb = pltpu.make_async_copy(src_b, dst_b, ld_sems.at[1])
            ca.start()
            cb.start()
            ca.wait()
            cb.wait()

        def add_hop(f, t):
            acc, rcv, _, _, _ = flow(f, t)

            def add_body(i, carry):
                j = i // (MC // MT)
                mi = lax.rem(i, MC // MT)
                row = pl.ds(mi * MT, MT)
                load2(acc.at[2 * t + j, row, :], rcv.at[2 * (t - 1) + j, row, :],
                      av, rv)
                cc[...] = (av[...].astype(jnp.float32)
                           + rv[...].astype(jnp.float32)).astype(COMM_DTYPE)
                local_copy(cc, acc.at[2 * t + j, row, :])
                return carry

            lax.fori_loop(0, (NH // NT) * (MC // MT), add_body, 0)

        def finish_hop(f, t):
            acc, rcv, col0, _, _ = flow(f, t)

            def fin_body(i, carry):
                j = i // (MC // MT)
                mi = lax.rem(i, MC // MT)
                row = pl.ds(mi * MT, MT)
                load2(acc.at[2 * t + j, row, :], rcv.at[2 * (t - 1) + j, row, :],
                      av, rv)
                cv[...] = _gelu(av[...].astype(jnp.float32)
                                + rv[...].astype(jnp.float32))
                local_copy(cv, out_ref.at[row, pl.ds(col0 + j * NT, NT)])
                return carry

            lax.fori_loop(0, (NH // NT) * (MC // MT), fin_body, 0)

        for f in range(2):
            _, _, _, _, chunk = flow(f, 0)
            load_chunk(chunk)
            compute_tiles(f, 0)
            send_hop(f, 0)

        def pre_body(t, carry):
            for f in range(2):
                _, _, _, _, chunk = flow(f, t)
                load_chunk(chunk)
                compute_tiles(f, t)
            return carry

        lax.fori_loop(1, N_DEV, pre_body, 0)

        for t in range(1, N_DEV - 1):
            for f in range(2):
                wait_hop(f, t - 1)
                add_hop(f, t)
                send_hop(f, t)

        t = N_DEV - 1
        for f in range(2):
            wait_hop(f, t - 1)
            finish_hop(f, t)

        for rdma in sends:
            rdma.wait_send()

    acc_buf = jax.ShapeDtypeStruct((N_DEV * 2, MC, NT), COMM_DTYPE)
    rcv_buf = jax.ShapeDtypeStruct(((N_DEV - 1) * 2, MC, NT), COMM_DTYPE)
    out, *_ = pl.pallas_call(
        body,
        out_shape=[
            jax.ShapeDtypeStruct((MC, N), jnp.float32),
            acc_buf,
            acc_buf,
            rcv_buf,
            rcv_buf,
        ],
        in_specs=[
            pl.BlockSpec(memory_space=pl.ANY),
            pl.BlockSpec(memory_space=pl.ANY),
        ],
        out_specs=[pl.BlockSpec(memory_space=pl.ANY)] * 5,
        scratch_shapes=[
            pltpu.VMEM((MT, K), jnp.float32),
            pltpu.VMEM((MC, K), jnp.bfloat16),
            pltpu.VMEM((K, NT), jnp.float32),
            pltpu.VMEM((MT, NT), jnp.float32),
            pltpu.VMEM((MT, NT), COMM_DTYPE),
            pltpu.VMEM((MT, NT), COMM_DTYPE),
            pltpu.VMEM((MT, NT), COMM_DTYPE),
            pltpu.SemaphoreType.DMA,
            pltpu.SemaphoreType.DMA((2,)),
            pltpu.SemaphoreType.DMA((2, N_DEV - 1)),
            pltpu.SemaphoreType.DMA((2, N_DEV - 1)),
        ],
        compiler_params=pltpu.CompilerParams(
            collective_id=0,
            vmem_limit_bytes=60 * 1024 * 1024,
        ),
    )(x, w_mat)
    return out
